# Initial kernel scaffold; baseline (speedup 1.0000x reference)
import jax
import jax.numpy as jnp
from jax.experimental import pallas as pl


def kernel(q_points, s_points, q_feats, s_feats, W1, b1, g1, be1, W2, b2, g2, be2):
    raise NotImplementedError("write your pallas kernel here")


# speedup vs baseline: 16.0583x; 16.0583x over previous
import functools

import jax
import jax.numpy as jnp
from jax import lax
from jax.experimental import pallas as pl
from jax.experimental.pallas import tpu as pltpu

B, Q, S, Cq, Cs = 8, 4096, 1024, 128, 256
D0, D1, D2 = Cq + Cs, 256, 128
QT = 256
NQT = Q // QT
N_BN = B * Q


def _kernel_a(qpt_ref, sp_ref, qf_ref, sf_ref, w1_ref, b1_ref,
              y1_ref, s1_ref, ss1_ref, p_scr):
    b = pl.program_id(0)
    qt = pl.program_id(1)

    @pl.when(qt == 0)
    def _():
        p_scr[...] = lax.dot_general(
            w1_ref[:, Cq:], sf_ref[0],
            (((1,), (0,)), ((), ())),
            preferred_element_type=jnp.float32)

    qpt = qpt_ref[0]
    sp = sp_ref[0]
    qq = jnp.sum(qpt * qpt, axis=1, keepdims=True)
    ss = jnp.sum(sp * sp, axis=0, keepdims=True)
    qs = lax.dot_general(qpt, sp, (((1,), (0,)), ((), ())),
                         preferred_element_type=jnp.float32)
    d2 = jnp.maximum(qq + ss - 2.0 * qs, 0.0)

    iota = lax.broadcasted_iota(jnp.int32, (QT, S), 1)
    dcur = d2
    dists = []
    idxs = []
    for _ in range(3):
        mk = jnp.min(dcur, axis=1, keepdims=True)
        ik = jnp.min(jnp.where(dcur == mk, iota, S), axis=1,
                     keepdims=True)
        dists.append(mk)
        idxs.append(ik)
        dcur = jnp.where(iota == ik, jnp.float32(jnp.inf), dcur)

    ws = [1.0 / (d + 1e-5) for d in dists]
    wsum = ws[0] + ws[1] + ws[2]
    ws = [w / wsum for w in ws]

    sel = jnp.zeros((QT, S), jnp.float32)
    for ik, wk in zip(idxs, ws):
        sel = sel + jnp.where(iota == ik, wk, 0.0)

    z = lax.dot_general(p_scr[...], sel, (((1,), (1,)), ((), ())),
                        preferred_element_type=jnp.float32)
    yq = lax.dot_general(w1_ref[:, :Cq], qf_ref[0],
                         (((1,), (0,)), ((), ())),
                         preferred_element_type=jnp.float32)
    y1 = yq + z + b1_ref[...]
    y1_ref[0] = y1

    @pl.when(jnp.logical_and(b == 0, qt == 0))
    def _():
        s1_ref[...] = jnp.zeros_like(s1_ref)
        ss1_ref[...] = jnp.zeros_like(ss1_ref)

    s1_ref[...] += jnp.sum(y1, axis=1, keepdims=True)
    ss1_ref[...] += jnp.sum(y1 * y1, axis=1, keepdims=True)


def _kernel_b(y1_ref, s1_ref, ss1_ref, g1_ref, be1_ref, w2_ref, b2_ref,
              y2_ref, s2_ref, ss2_ref):
    b = pl.program_id(0)
    qt = pl.program_id(1)

    mean = s1_ref[...] * (1.0 / N_BN)
    var = ss1_ref[...] * (1.0 / N_BN) - mean * mean
    scale = g1_ref[...] * lax.rsqrt(var + 1e-5)
    shift = be1_ref[...] - mean * scale
    x1 = jnp.maximum(y1_ref[0] * scale + shift, 0.0)
    y2 = lax.dot_general(w2_ref[...], x1, (((1,), (0,)), ((), ())),
                         preferred_element_type=jnp.float32) + b2_ref[...]
    y2_ref[0] = y2

    @pl.when(jnp.logical_and(b == 0, qt == 0))
    def _():
        s2_ref[...] = jnp.zeros_like(s2_ref)
        ss2_ref[...] = jnp.zeros_like(ss2_ref)

    s2_ref[...] += jnp.sum(y2, axis=1, keepdims=True)
    ss2_ref[...] += jnp.sum(y2 * y2, axis=1, keepdims=True)


def _kernel_c(y2_ref, s2_ref, ss2_ref, g2_ref, be2_ref, out_ref):
    mean = s2_ref[...] * (1.0 / N_BN)
    var = ss2_ref[...] * (1.0 / N_BN) - mean * mean
    scale = g2_ref[...] * lax.rsqrt(var + 1e-5)
    shift = be2_ref[...] - mean * scale
    out_ref[0] = jnp.maximum(y2_ref[0] * scale + shift, 0.0)


def kernel(q_points, s_points, q_feats, s_feats, W1, b1, g1, be1,
           W2, b2, g2, be2):
    qpt = q_points.transpose(0, 2, 1)
    b1c = b1.reshape(D1, 1)
    g1c = g1.reshape(D1, 1)
    be1c = be1.reshape(D1, 1)
    b2c = b2.reshape(D2, 1)
    g2c = g2.reshape(D2, 1)
    be2c = be2.reshape(D2, 1)

    col = lambda d: pl.BlockSpec((d, 1), lambda b, q: (0, 0))
    y1, s1, ss1 = pl.pallas_call(
        _kernel_a,
        grid=(B, NQT),
        in_specs=[
            pl.BlockSpec((1, QT, 3), lambda b, q: (b, q, 0)),
            pl.BlockSpec((1, 3, S), lambda b, q: (b, 0, 0)),
            pl.BlockSpec((1, Cq, QT), lambda b, q: (b, 0, q)),
            pl.BlockSpec((1, Cs, S), lambda b, q: (b, 0, 0)),
            pl.BlockSpec((D1, D0), lambda b, q: (0, 0)),
            col(D1),
        ],
        out_specs=[
            pl.BlockSpec((1, D1, QT), lambda b, q: (b, 0, q)),
            col(D1),
            col(D1),
        ],
        out_shape=[
            jax.ShapeDtypeStruct((B, D1, Q), jnp.float32),
            jax.ShapeDtypeStruct((D1, 1), jnp.float32),
            jax.ShapeDtypeStruct((D1, 1), jnp.float32),
        ],
        scratch_shapes=[pltpu.VMEM((D1, S), jnp.float32)],
    )(qpt, s_points, q_feats, s_feats, W1, b1c)

    y2, s2, ss2 = pl.pallas_call(
        _kernel_b,
        grid=(B, NQT),
        in_specs=[
            pl.BlockSpec((1, D1, QT), lambda b, q: (b, 0, q)),
            col(D1), col(D1), col(D1), col(D1),
            pl.BlockSpec((D2, D1), lambda b, q: (0, 0)),
            col(D2),
        ],
        out_specs=[
            pl.BlockSpec((1, D2, QT), lambda b, q: (b, 0, q)),
            col(D2),
            col(D2),
        ],
        out_shape=[
            jax.ShapeDtypeStruct((B, D2, Q), jnp.float32),
            jax.ShapeDtypeStruct((D2, 1), jnp.float32),
            jax.ShapeDtypeStruct((D2, 1), jnp.float32),
        ],
    )(y1, s1, ss1, g1c, be1c, W2, b2c)

    out = pl.pallas_call(
        _kernel_c,
        grid=(B, NQT),
        in_specs=[
            pl.BlockSpec((1, D2, QT), lambda b, q: (b, 0, q)),
            col(D2), col(D2), col(D2), col(D2),
        ],
        out_specs=pl.BlockSpec((1, D2, QT), lambda b, q: (b, 0, q)),
        out_shape=jax.ShapeDtypeStruct((B, D2, Q), jnp.float32),
    )(y2, s2, ss2, g2c, be2c)
    return out


# speedup vs baseline: 19.3503x; 1.2050x over previous
import functools

import jax
import jax.numpy as jnp
from jax import lax
from jax.experimental import pallas as pl
from jax.experimental.pallas import tpu as pltpu

B, Q, S, Cq, Cs = 8, 4096, 1024, 128, 256
D0, D1, D2 = Cq + Cs, 256, 128
QT = 256
NQT = Q // QT
N_BN = B * Q


def _kernel_a(qpt_ref, sp_ref, qf_ref, sf_ref, w1_ref, b1_ref,
              y1_ref, s1_ref, ss1_ref, p_scr):
    b = pl.program_id(0)
    qt = pl.program_id(1)

    @pl.when(qt == 0)
    def _():
        p_scr[...] = lax.dot_general(
            w1_ref[:, Cq:], sf_ref[0],
            (((1,), (0,)), ((), ())),
            preferred_element_type=jnp.float32)

    qpt = qpt_ref[0]
    sp = sp_ref[0]
    qq = jnp.sum(qpt * qpt, axis=1, keepdims=True)
    ss = jnp.sum(sp * sp, axis=0, keepdims=True)
    qs = lax.dot_general(qpt, sp, (((1,), (0,)), ((), ())),
                         preferred_element_type=jnp.float32)
    d2 = qq + ss - 2.0 * qs

    inf = jnp.float32(jnp.inf)
    m1 = jnp.min(d2, axis=1, keepdims=True)
    m2 = jnp.min(jnp.where(d2 > m1, d2, inf), axis=1, keepdims=True)
    m3 = jnp.min(jnp.where(d2 > m2, d2, inf), axis=1, keepdims=True)

    ws = [1.0 / (jnp.maximum(m, 0.0) + 1e-5) for m in (m1, m2, m3)]
    wsum = ws[0] + ws[1] + ws[2]
    ws = [w / wsum for w in ws]

    sel = jnp.where(
        d2 == m1, ws[0],
        jnp.where(d2 == m2, ws[1], jnp.where(d2 == m3, ws[2], 0.0)))

    z = lax.dot_general(p_scr[...], sel, (((1,), (1,)), ((), ())),
                        preferred_element_type=jnp.float32)
    yq = lax.dot_general(w1_ref[:, :Cq], qf_ref[0],
                         (((1,), (0,)), ((), ())),
                         preferred_element_type=jnp.float32)
    y1 = yq + z + b1_ref[...]
    y1_ref[0] = y1

    @pl.when(jnp.logical_and(b == 0, qt == 0))
    def _():
        s1_ref[...] = jnp.zeros_like(s1_ref)
        ss1_ref[...] = jnp.zeros_like(ss1_ref)

    s1_ref[...] += jnp.sum(y1, axis=1, keepdims=True)
    ss1_ref[...] += jnp.sum(y1 * y1, axis=1, keepdims=True)


def _kernel_b(y1_ref, s1_ref, ss1_ref, g1_ref, be1_ref, w2_ref, b2_ref,
              y2_ref, s2_ref, ss2_ref):
    b = pl.program_id(0)
    qt = pl.program_id(1)

    mean = s1_ref[...] * (1.0 / N_BN)
    var = ss1_ref[...] * (1.0 / N_BN) - mean * mean
    scale = g1_ref[...] * lax.rsqrt(var + 1e-5)
    shift = be1_ref[...] - mean * scale
    x1 = jnp.maximum(y1_ref[0] * scale + shift, 0.0)
    y2 = lax.dot_general(w2_ref[...], x1, (((1,), (0,)), ((), ())),
                         preferred_element_type=jnp.float32) + b2_ref[...]
    y2_ref[0] = y2

    @pl.when(jnp.logical_and(b == 0, qt == 0))
    def _():
        s2_ref[...] = jnp.zeros_like(s2_ref)
        ss2_ref[...] = jnp.zeros_like(ss2_ref)

    s2_ref[...] += jnp.sum(y2, axis=1, keepdims=True)
    ss2_ref[...] += jnp.sum(y2 * y2, axis=1, keepdims=True)


def _kernel_c(y2_ref, s2_ref, ss2_ref, g2_ref, be2_ref, out_ref):
    mean = s2_ref[...] * (1.0 / N_BN)
    var = ss2_ref[...] * (1.0 / N_BN) - mean * mean
    scale = g2_ref[...] * lax.rsqrt(var + 1e-5)
    shift = be2_ref[...] - mean * scale
    out_ref[0] = jnp.maximum(y2_ref[0] * scale + shift, 0.0)


def kernel(q_points, s_points, q_feats, s_feats, W1, b1, g1, be1,
           W2, b2, g2, be2):
    qpt = q_points.transpose(0, 2, 1)
    b1c = b1.reshape(D1, 1)
    g1c = g1.reshape(D1, 1)
    be1c = be1.reshape(D1, 1)
    b2c = b2.reshape(D2, 1)
    g2c = g2.reshape(D2, 1)
    be2c = be2.reshape(D2, 1)

    col = lambda d: pl.BlockSpec((d, 1), lambda b, q: (0, 0))
    y1, s1, ss1 = pl.pallas_call(
        _kernel_a,
        grid=(B, NQT),
        in_specs=[
            pl.BlockSpec((1, QT, 3), lambda b, q: (b, q, 0)),
            pl.BlockSpec((1, 3, S), lambda b, q: (b, 0, 0)),
            pl.BlockSpec((1, Cq, QT), lambda b, q: (b, 0, q)),
            pl.BlockSpec((1, Cs, S), lambda b, q: (b, 0, 0)),
            pl.BlockSpec((D1, D0), lambda b, q: (0, 0)),
            col(D1),
        ],
        out_specs=[
            pl.BlockSpec((1, D1, QT), lambda b, q: (b, 0, q)),
            col(D1),
            col(D1),
        ],
        out_shape=[
            jax.ShapeDtypeStruct((B, D1, Q), jnp.float32),
            jax.ShapeDtypeStruct((D1, 1), jnp.float32),
            jax.ShapeDtypeStruct((D1, 1), jnp.float32),
        ],
        scratch_shapes=[pltpu.VMEM((D1, S), jnp.float32)],
    )(qpt, s_points, q_feats, s_feats, W1, b1c)

    y2, s2, ss2 = pl.pallas_call(
        _kernel_b,
        grid=(B, NQT),
        in_specs=[
            pl.BlockSpec((1, D1, QT), lambda b, q: (b, 0, q)),
            col(D1), col(D1), col(D1), col(D1),
            pl.BlockSpec((D2, D1), lambda b, q: (0, 0)),
            col(D2),
        ],
        out_specs=[
            pl.BlockSpec((1, D2, QT), lambda b, q: (b, 0, q)),
            col(D2),
            col(D2),
        ],
        out_shape=[
            jax.ShapeDtypeStruct((B, D2, Q), jnp.float32),
            jax.ShapeDtypeStruct((D2, 1), jnp.float32),
            jax.ShapeDtypeStruct((D2, 1), jnp.float32),
        ],
    )(y1, s1, ss1, g1c, be1c, W2, b2c)

    out = pl.pallas_call(
        _kernel_c,
        grid=(B, NQT),
        in_specs=[
            pl.BlockSpec((1, D2, QT), lambda b, q: (b, 0, q)),
            col(D2), col(D2), col(D2), col(D2),
        ],
        out_specs=pl.BlockSpec((1, D2, QT), lambda b, q: (b, 0, q)),
        out_shape=jax.ShapeDtypeStruct((B, D2, Q), jnp.float32),
    )(y2, s2, ss2, g2c, be2c)
    return out
